# baseline (speedup 1.0000x reference)
import functools

import jax
import jax.numpy as jnp
from jax import lax
from jax.experimental import pallas as pl
from jax.experimental.pallas import tpu as pltpu
from jax.experimental.pallas import tpu_sc as plsc

NC = 2
NS = 16
NW = NC * NS
CH = 16
LANES = 16
ND = 4
NE = 2


def _make_sc_call(n_rows, d, emb_rows):
  per_w = (n_rows // CH) // NW
  vregs = d // LANES
  rows_per_tile = emb_rows // NS
  mesh = plsc.VectorSubcoreMesh(
      core_axis_name="c", subcore_axis_name="s",
      num_cores=NC, num_subcores=NS)

  @functools.partial(
      pl.kernel,
      out_type=jax.ShapeDtypeStruct((n_rows, d), jnp.float32),
      mesh=mesh,
      scratch_types=(
          [pltpu.VMEM((CH * d,), jnp.float32)] * NE
          + [pltpu.VMEM((CH, d), jnp.float32)] * ND
          + [pltpu.VMEM((per_w * CH,), jnp.int32)]
          + [pltpu.VMEM_SHARED((emb_rows * d,), jnp.float32)]
          + [pltpu.SemaphoreType.DMA] * (NE + 2 * ND)
      ),
  )
  def run(data_h, pos_h, emb_h, out_h, *bufs):
    ebufs = bufs[:NE]
    dbufs = bufs[NE:NE + ND]
    idxs = bufs[NE + ND]
    stab = bufs[NE + ND + 1]
    gsems = bufs[NE + ND + 2:NE + ND + 2 + NE]
    isems = bufs[NE + ND + 2 + NE:NE + ND + 2 + NE + ND]
    osems = bufs[NE + ND + 2 + NE + ND:]

    wid = lax.axis_index("s") * NC + lax.axis_index("c")
    t0 = wid * per_w

    def row0(t_rel):
      return (t0 + t_rel) * CH

    def add_into(dst, src):
      @pl.loop(0, CH)
      def _(r):
        rb = r * d
        for k in range(vregs):
          plsc.addupdate(dst.at[r, pl.ds(k * LANES, LANES)],
                         src[pl.ds(rb + k * LANES, LANES)])

    def issue_gather(t_rel, e):
      off = idxs[pl.ds(t_rel * CH, LANES)][0]
      pltpu.async_copy(stab.at[pl.ds(pl.multiple_of(off * d, 8), CH * d)],
                       ebufs[e], gsems[e])

    def wait_gather(e):
      pltpu.make_async_copy(stab.at[pl.ds(0, CH * d)],
                            ebufs[e], gsems[e]).wait()

    def issue_in(t_rel, s):
      pltpu.async_copy(data_h.at[pl.ds(row0(t_rel), CH)], dbufs[s],
                       isems[s])

    def wait_in(s):
      pltpu.make_async_copy(data_h.at[pl.ds(0, CH)], dbufs[s],
                            isems[s]).wait()

    def issue_out(t_rel, s):
      pltpu.async_copy(dbufs[s], out_h.at[pl.ds(row0(t_rel), CH)],
                       osems[s])

    def wait_out(s):
      pltpu.make_async_copy(dbufs[s], out_h.at[pl.ds(0, CH)],
                            osems[s]).wait()

    sid = lax.axis_index("s")
    for p in range(rows_per_tile // CH):
      er0 = (sid * rows_per_tile + p * CH) * d
      pltpu.sync_copy(emb_h.at[pl.ds(er0, CH * d)], ebufs[0])
      pltpu.sync_copy(ebufs[0], stab.at[pl.ds(er0, CH * d)])
    plsc.subcore_barrier()

    pltpu.sync_copy(pos_h.at[pl.ds(t0 * CH, per_w * CH)], idxs)
    issue_gather(0, 0)
    issue_in(0, 0)
    issue_in(1, 1)

    def body(t_rel, e, s, t_ge2, has1, has2):
      wait_gather(e)
      wait_in(s)
      add_into(dbufs[s], ebufs[e])
      if has1:
        issue_gather(t_rel + 1, (e + 1) % NE)
      issue_out(t_rel, s)
      if t_ge2:
        wait_out((s + 2) % ND)
      if has2:
        issue_in(t_rel + 2, (s + 2) % ND)

    body(0, 0, 0, False, True, True)
    body(1, 1, 1, False, True, True)

    @pl.loop(2, per_w - 2, step=ND)
    def _(t):
      for u in range(ND):
        body(t + u, (2 + u) % NE, (2 + u) % ND, True, True, True)

    body(per_w - 2, (per_w - 2) % NE, (per_w - 2) % ND, True, True, False)
    body(per_w - 1, (per_w - 1) % NE, (per_w - 1) % ND, True, False, False)
    wait_out((per_w - 2) % ND)
    wait_out((per_w - 1) % ND)

  return run


def _tc_dense_call(b, l, d, pad_pos):
  def body(lens_ref, lang_ref, f_ref, emb_ref, out_l_ref, out_f_ref):
    i = pl.program_id(0)
    out_l_ref[...] = lang_ref[...] + emb_ref[:l, :][None]
    off = lens_ref[i]
    base = pl.multiple_of((off // 8) * 8, 8)
    r = off - base
    sl = emb_ref[pl.ds(base, l + 8), :]
    rolled = pltpu.roll(sl, jnp.where(r == 0, 0, l + 8 - r), 0)
    out_f_ref[...] = f_ref[...] + rolled[:l, :][None]

  spec = pl.BlockSpec((1, l, d), lambda i: (i, 0, 0))
  return pl.pallas_call(
      body,
      out_shape=(jax.ShapeDtypeStruct((b, l, d), jnp.float32),) * 2,
      grid=(b,),
      in_specs=[
          pl.BlockSpec(memory_space=pltpu.SMEM),
          spec,
          spec,
          pl.BlockSpec((pad_pos, d), lambda i: (0, 0)),
      ],
      out_specs=(spec, spec),
  )


def kernel(lang, frames, actions, lens_lang, lens_frames, emb):
  b, l, d = lang.shape
  n_rows = b * l
  lens32 = lens_lang.astype(jnp.int32)

  pos_a = (jnp.arange(l, dtype=jnp.int32)[None, :]
           + lens32[:, None]).reshape(-1)

  pad_pos = ((l - 1) // 8 + 1) * 8 + l + 8
  assert pad_pos <= emb.shape[0]

  emb_rows = ((l + l - 1) // CH + 1) * CH
  out_a = _make_sc_call(n_rows, d, emb_rows)(
      actions.reshape(n_rows, d), pos_a, emb.reshape(-1))
  out_l, out_f = _tc_dense_call(b, l, d, pad_pos)(lens32, lang, frames, emb)
  return (out_l, out_f, out_a.reshape(b, l, d))


# speedup vs baseline: 1.0184x; 1.0184x over previous
import functools

import jax
import jax.numpy as jnp
from jax import lax
from jax.experimental import pallas as pl
from jax.experimental.pallas import tpu as pltpu
from jax.experimental.pallas import tpu_sc as plsc

NC = 2
NS = 16
NW = NC * NS
CH = 16
LANES = 16
ND = 4
NE = 2


def _make_sc_call(n_rows, d, emb_rows):
  per_w = (n_rows // CH) // NW
  vregs = d // LANES
  rows_per_tile = emb_rows // NS
  mesh = plsc.VectorSubcoreMesh(
      core_axis_name="c", subcore_axis_name="s",
      num_cores=NC, num_subcores=NS)

  @functools.partial(
      pl.kernel,
      out_type=jax.ShapeDtypeStruct((n_rows, d), jnp.float32),
      mesh=mesh,
      scratch_types=(
          [pltpu.VMEM((CH * d,), jnp.float32)] * NE
          + [pltpu.VMEM((CH, d), jnp.float32)] * ND
          + [pltpu.VMEM((per_w * CH,), jnp.int32)]
          + [pltpu.VMEM_SHARED((emb_rows * d,), jnp.float32)]
          + [pltpu.SemaphoreType.DMA] * (NE + 2 * ND)
      ),
  )
  def run(data_h, pos_h, emb_h, out_h, *bufs):
    ebufs = bufs[:NE]
    dbufs = bufs[NE:NE + ND]
    idxs = bufs[NE + ND]
    stab = bufs[NE + ND + 1]
    gsems = bufs[NE + ND + 2:NE + ND + 2 + NE]
    isems = bufs[NE + ND + 2 + NE:NE + ND + 2 + NE + ND]
    osems = bufs[NE + ND + 2 + NE + ND:]

    wid = lax.axis_index("s") * NC + lax.axis_index("c")
    t0 = wid * per_w

    def row0(t_rel):
      return (t0 + t_rel) * CH

    def add_into(dst, src):
      @pl.loop(0, CH)
      def _(r):
        rb = r * d
        for k in range(vregs):
          plsc.addupdate(dst.at[r, pl.ds(k * LANES, LANES)],
                         src[pl.ds(rb + k * LANES, LANES)])

    def issue_gather(t_rel, e):
      off = idxs[pl.ds(t_rel * CH, LANES)][0]
      pltpu.async_copy(stab.at[pl.ds(pl.multiple_of(off * d, 8), CH * d)],
                       ebufs[e], gsems[e])

    def wait_gather(e):
      pltpu.make_async_copy(stab.at[pl.ds(0, CH * d)],
                            ebufs[e], gsems[e]).wait()

    def issue_in(t_rel, s):
      pltpu.async_copy(data_h.at[pl.ds(row0(t_rel), CH)], dbufs[s],
                       isems[s])

    def wait_in(s):
      pltpu.make_async_copy(data_h.at[pl.ds(0, CH)], dbufs[s],
                            isems[s]).wait()

    def issue_out(t_rel, s):
      pltpu.async_copy(dbufs[s], out_h.at[pl.ds(row0(t_rel), CH)],
                       osems[s])

    def wait_out(s):
      pltpu.make_async_copy(dbufs[s], out_h.at[pl.ds(0, CH)],
                            osems[s]).wait()

    sid = lax.axis_index("s")
    for p in range(rows_per_tile // CH):
      er0 = (sid * rows_per_tile + p * CH) * d
      pltpu.sync_copy(emb_h.at[pl.ds(er0, CH * d)], ebufs[0])
      pltpu.sync_copy(ebufs[0], stab.at[pl.ds(er0, CH * d)])
    plsc.subcore_barrier()

    pltpu.sync_copy(pos_h.at[pl.ds(t0 * CH, per_w * CH)], idxs)
    issue_gather(0, 0)
    issue_in(0, 0)
    issue_in(1, 1)

    def body(t_rel, e, s, t_ge2, has1, has2):
      wait_gather(e)
      wait_in(s)
      add_into(dbufs[s], ebufs[e])
      if has1:
        issue_gather(t_rel + 1, (e + 1) % NE)
      issue_out(t_rel, s)
      if t_ge2:
        wait_out((s + 2) % ND)
      if has2:
        issue_in(t_rel + 2, (s + 2) % ND)

    body(0, 0, 0, False, True, True)
    body(1, 1, 1, False, True, True)

    @pl.loop(2, per_w - 2, step=ND)
    def _(t):
      for u in range(ND):
        body(t + u, (2 + u) % NE, (2 + u) % ND, True, True, True)

    body(per_w - 2, (per_w - 2) % NE, (per_w - 2) % ND, True, True, False)
    body(per_w - 1, (per_w - 1) % NE, (per_w - 1) % ND, True, False, False)
    wait_out((per_w - 2) % ND)
    wait_out((per_w - 1) % ND)

  return run


def _tc_lang_call(b, l, d):
  def body(lang_ref, emb_ref, out_ref):
    out_ref[...] = lang_ref[...] + emb_ref[...][None]

  return pl.pallas_call(
      body,
      out_shape=jax.ShapeDtypeStruct((b, l, d), jnp.float32),
      grid=(b,),
      in_specs=[
          pl.BlockSpec((1, l, d), lambda i: (i, 0, 0)),
          pl.BlockSpec((l, d), lambda i: (0, 0)),
      ],
      out_specs=pl.BlockSpec((1, l, d), lambda i: (i, 0, 0)),
  )


def _tc_frames_call(b, l, d, pad_pos):
  def body(lens_ref, f_ref, emb_ref, out_ref):
    i = pl.program_id(0)
    off = lens_ref[i]
    base = pl.multiple_of((off // 8) * 8, 8)
    r = off - base
    sl = emb_ref[pl.ds(base, l + 8), :]
    rolled = pltpu.roll(sl, jnp.where(r == 0, 0, l + 8 - r), 0)
    out_ref[...] = f_ref[...] + rolled[:l, :][None]

  return pl.pallas_call(
      body,
      out_shape=jax.ShapeDtypeStruct((b, l, d), jnp.float32),
      grid=(b,),
      in_specs=[
          pl.BlockSpec(memory_space=pltpu.SMEM),
          pl.BlockSpec((1, l, d), lambda i: (i, 0, 0)),
          pl.BlockSpec((pad_pos, d), lambda i: (0, 0)),
      ],
      out_specs=pl.BlockSpec((1, l, d), lambda i: (i, 0, 0)),
  )


def kernel(lang, frames, actions, lens_lang, lens_frames, emb):
  b, l, d = lang.shape
  n_rows = b * l
  lens32 = lens_lang.astype(jnp.int32)

  pos_a = (jnp.arange(l, dtype=jnp.int32)[None, :]
           + lens32[:, None]).reshape(-1)

  pad_pos = ((l - 1) // 8 + 1) * 8 + l + 8
  emb_pad = jnp.pad(emb, ((0, max(0, pad_pos - emb.shape[0])), (0, 0)))

  emb_rows = ((l + l - 1) // CH + 1) * CH
  emb_flat = lax.slice(emb, (0, 0), (emb_rows, d)).reshape(emb_rows * d)
  out_a = _make_sc_call(n_rows, d, emb_rows)(
      actions.reshape(n_rows, d), pos_a, emb_flat)
  out_l = _tc_lang_call(b, l, d)(lang, lax.slice(emb, (0, 0), (l, d)))
  out_f = _tc_frames_call(b, l, d, pad_pos)(lens32, frames, emb_pad)
  return (out_l, out_f, out_a.reshape(b, l, d))
